# Initial kernel scaffold; baseline (speedup 1.0000x reference)
import jax
import jax.numpy as jnp
from jax.experimental import pallas as pl


def kernel(q, k, v, Wq, Wk, Wv, Wo):
    raise NotImplementedError("write your pallas kernel here")


# speedup vs baseline: 3.2247x; 3.2247x over previous
import functools

import jax
import jax.numpy as jnp
from jax.experimental import pallas as pl

HEADS = 12
DQK = 64
DV = 64
SPAN = 32


def _mha_band_kernel(q_ref, k_ref, v_ref, wq_ref, wk_ref, wv_ref, wo_ref,
                     out_ref):
    h = pl.program_id(0)
    dot = functools.partial(jax.lax.dot_general,
                            precision=jax.lax.Precision.HIGHEST,
                            preferred_element_type=jnp.float32)

    qm = q_ref[...]
    km = k_ref[...]
    vm = v_ref[...]
    wq = wq_ref[0]
    wk = wk_ref[0]
    wv = wv_ref[0]
    wo = wo_ref[0]

    lq = qm.shape[0]

    qh = dot(qm, wq, dimension_numbers=(((1,), (1,)), ((), ())))
    kh = dot(km, wk, dimension_numbers=(((1,), (1,)), ((), ())))
    vh = dot(vm, wv, dimension_numbers=(((1,), (1,)), ((), ())))

    pad = jnp.zeros((SPAN - 1, DQK), jnp.float32)
    kpad = jnp.concatenate([pad, kh], axis=0)
    vpad = jnp.concatenate([pad, vh], axis=0)

    scale = 1.0 / (DQK ** 0.5)
    cols = []
    for i in range(SPAN):
        ks = jax.lax.slice_in_dim(kpad, i, i + lq, axis=0)
        cols.append(jnp.sum(qh * ks, axis=1, keepdims=True) * scale)
    s = jnp.concatenate(cols, axis=1)

    jidx = jax.lax.broadcasted_iota(jnp.int32, (lq, SPAN), 0)
    iidx = jax.lax.broadcasted_iota(jnp.int32, (lq, SPAN), 1)
    valid = jidx + iidx >= SPAN - 1
    s = jnp.where(valid, s, -jnp.inf)

    m = jnp.max(s, axis=0, keepdims=True)
    e = jnp.exp(s - m)
    w = e / jnp.sum(e, axis=0, keepdims=True)

    acc = jnp.zeros((lq, DV), jnp.float32)
    for i in range(SPAN):
        vs = jax.lax.slice_in_dim(vpad, i, i + lq, axis=0)
        acc = acc + w[:, i:i + 1] * vs

    partial = dot(acc, wo, dimension_numbers=(((1,), (1,)), ((), ())))

    @pl.when(h == 0)
    def _():
        out_ref[...] = partial

    @pl.when(h != 0)
    def _():
        out_ref[...] += partial


def kernel(q, k, v, Wq, Wk, Wv, Wo):
    b, lq, dim_q = q.shape
    lkv = k.shape[1]
    dim_out = Wo.shape[0]

    q2 = q.reshape(lq, dim_q)
    k2 = k.reshape(lkv, k.shape[2])
    v2 = v.reshape(lkv, v.shape[2])
    wq3 = Wq.reshape(HEADS, DQK, dim_q)
    wk3 = Wk.reshape(HEADS, DQK, k.shape[2])
    wv3 = Wv.reshape(HEADS, DV, v.shape[2])
    wo3 = Wo.reshape(dim_out, HEADS, DV).transpose(1, 0, 2)

    out = pl.pallas_call(
        _mha_band_kernel,
        grid=(HEADS,),
        in_specs=[
            pl.BlockSpec((lq, dim_q), lambda h: (0, 0)),
            pl.BlockSpec((lkv, k.shape[2]), lambda h: (0, 0)),
            pl.BlockSpec((lkv, v.shape[2]), lambda h: (0, 0)),
            pl.BlockSpec((1, DQK, dim_q), lambda h: (h, 0, 0)),
            pl.BlockSpec((1, DQK, k.shape[2]), lambda h: (h, 0, 0)),
            pl.BlockSpec((1, DV, v.shape[2]), lambda h: (h, 0, 0)),
            pl.BlockSpec((1, dim_out, DV), lambda h: (h, 0, 0)),
        ],
        out_specs=pl.BlockSpec((lq, dim_out), lambda h: (0, 0)),
        out_shape=jax.ShapeDtypeStruct((lq, dim_out), jnp.float32),
    )(q2, k2, v2, wq3, wk3, wv3, wo3)

    return out.reshape(b, lq, dim_out)


# speedup vs baseline: 13.9223x; 4.3174x over previous
import functools

import jax
import jax.numpy as jnp
from jax.experimental import pallas as pl
from jax.experimental.pallas import tpu as pltpu

HEADS = 12
DQK = 64
DV = 64
SPAN = 32
LQ = 2048
DIM = 768

_dot = functools.partial(jax.lax.dot_general,
                         preferred_element_type=jnp.float32)


def _proj_kernel(q_ref, k_ref, v_ref, wq_ref, wk_ref, wv_ref,
                 qt_ref, kt_ref, vt_ref):
    dn = (((1,), (1,)), ((), ()))
    qt_ref[...] = _dot(wq_ref[...], q_ref[...], dimension_numbers=dn)
    kt_ref[...] = _dot(wk_ref[...], k_ref[...], dimension_numbers=dn)
    vt_ref[...] = _dot(wv_ref[...], v_ref[...], dimension_numbers=dn)


def _band_kernel(qt_ref, kt_ref, vt_ref, wo_ref, out_ref, qkvt_ref):
    h = pl.program_id(0)
    qt = qt_ref[...]
    kt = kt_ref[...]
    vt = vt_ref[...]

    zpad = jnp.zeros((DQK, SPAN - 1), jnp.float32)
    ktp = jnp.concatenate([zpad, kt], axis=1)
    vtp = jnp.concatenate([zpad, vt], axis=1)

    scale = 1.0 / (DQK ** 0.5)
    rows = []
    for i in range(SPAN):
        ks = jax.lax.slice_in_dim(ktp, i, i + LQ, axis=1)
        rows.append(jnp.sum(qt * ks, axis=0, keepdims=True))
    s = jnp.concatenate(rows, axis=0) * scale

    iidx = jax.lax.broadcasted_iota(jnp.int32, (SPAN, LQ), 0)
    jidx = jax.lax.broadcasted_iota(jnp.int32, (SPAN, LQ), 1)
    s = jnp.where(iidx + jidx >= SPAN - 1, s, -jnp.inf)

    m = jnp.max(s, axis=1, keepdims=True)
    e = jnp.exp(s - m)
    w = e / jnp.sum(e, axis=1, keepdims=True)

    acc = jnp.zeros((DV, LQ), jnp.float32)
    for i in range(SPAN):
        vs = jax.lax.slice_in_dim(vtp, i, i + LQ, axis=1)
        acc = acc + w[i:i + 1, :] * vs

    qkvt_ref[pl.ds(h * DV, DV), :] = acc

    @pl.when(h == HEADS - 1)
    def _():
        out_ref[...] = _dot(qkvt_ref[...], wo_ref[...],
                            dimension_numbers=(((0,), (1,)), ((), ())))


def kernel(q, k, v, Wq, Wk, Wv, Wo):
    b, lq, dim_q = q.shape
    q2 = q.reshape(lq, dim_q)
    k2 = k.reshape(lq, dim_q)
    v2 = v.reshape(lq, dim_q)

    qt, kt, vt = pl.pallas_call(
        _proj_kernel,
        grid=(1,),
        in_specs=[pl.BlockSpec((LQ, DIM), lambda i: (0, 0))] * 3
        + [pl.BlockSpec((DIM, DIM), lambda i: (0, 0))] * 3,
        out_specs=[pl.BlockSpec((DIM, LQ), lambda i: (0, 0))] * 3,
        out_shape=[jax.ShapeDtypeStruct((DIM, LQ), jnp.float32)] * 3,
    )(q2, k2, v2, Wq, Wk, Wv)

    out = pl.pallas_call(
        _band_kernel,
        grid=(HEADS,),
        in_specs=[
            pl.BlockSpec((DQK, LQ), lambda h: (h, 0)),
            pl.BlockSpec((DQK, LQ), lambda h: (h, 0)),
            pl.BlockSpec((DV, LQ), lambda h: (h, 0)),
            pl.BlockSpec((DIM, HEADS * DV), lambda h: (0, 0)),
        ],
        out_specs=pl.BlockSpec((LQ, DIM), lambda h: (0, 0)),
        out_shape=jax.ShapeDtypeStruct((LQ, DIM), jnp.float32),
        scratch_shapes=[pltpu.VMEM((HEADS * DV, LQ), jnp.float32)],
    )(qt, kt, vt, Wo)

    return out.reshape(b, lq, DIM)


# speedup vs baseline: 14.7272x; 1.0578x over previous
import functools

import jax
import jax.numpy as jnp
from jax.experimental import pallas as pl
from jax.experimental.pallas import tpu as pltpu

HEADS = 12
DQK = 64
DV = 64
SPAN = 32
LQ = 2048
DIM = 768

_dot = functools.partial(jax.lax.dot_general,
                         preferred_element_type=jnp.float32)


def _proj_kernel(q_ref, k_ref, v_ref, wq_ref, wk_ref, wv_ref,
                 qt_ref, kt_ref, vt_ref):
    dn = (((1,), (1,)), ((), ()))
    qt_ref[...] = _dot(wq_ref[...], q_ref[...], dimension_numbers=dn)
    kt_ref[...] = _dot(wk_ref[...], k_ref[...], dimension_numbers=dn)
    vt_ref[...] = _dot(wv_ref[...], v_ref[...], dimension_numbers=dn)


_BLK = 128
_NBLK = LQ // _BLK


def _band_kernel(qt_ref, kt_ref, vt_ref, wo_ref, out_ref, qkvt_ref):
    h = pl.program_id(0)
    qt = qt_ref[...]
    kt = kt_ref[...]
    vt = vt_ref[...]

    zpad = jnp.zeros((DQK, SPAN - 1), jnp.float32)
    ktp = jnp.concatenate([zpad, kt], axis=1)
    vtp = jnp.concatenate([zpad, vt], axis=1)

    scale = 1.0 / (DQK ** 0.5)
    s_blocks = []
    for t in range(_NBLK):
        qtb = jax.lax.slice_in_dim(qt, t * _BLK, (t + 1) * _BLK, axis=1)
        rows = []
        for i in range(SPAN):
            ks = jax.lax.slice_in_dim(ktp, t * _BLK + i, t * _BLK + i + _BLK,
                                      axis=1)
            rows.append(jnp.sum(qtb * ks, axis=0, keepdims=True))
        s_blocks.append(jnp.concatenate(rows, axis=0))
    s = jnp.concatenate(s_blocks, axis=1) * scale

    iidx = jax.lax.broadcasted_iota(jnp.int32, (SPAN, LQ), 0)
    jidx = jax.lax.broadcasted_iota(jnp.int32, (SPAN, LQ), 1)
    s = jnp.where(iidx + jidx >= SPAN - 1, s, -jnp.inf)

    m = jnp.max(s, axis=1, keepdims=True)
    e = jnp.exp(s - m)
    w = e / jnp.sum(e, axis=1, keepdims=True)

    for t in range(_NBLK):
        accb = jnp.zeros((DV, _BLK), jnp.float32)
        wb = jax.lax.slice_in_dim(w, t * _BLK, (t + 1) * _BLK, axis=1)
        for i in range(SPAN):
            vs = jax.lax.slice_in_dim(vtp, t * _BLK + i, t * _BLK + i + _BLK,
                                      axis=1)
            accb = accb + wb[i:i + 1, :] * vs
        qkvt_ref[pl.ds(h * DV, DV), t * _BLK:(t + 1) * _BLK] = accb

    @pl.when(h == HEADS - 1)
    def _():
        out_ref[...] = _dot(qkvt_ref[...], wo_ref[...],
                            dimension_numbers=(((0,), (1,)), ((), ())))


def kernel(q, k, v, Wq, Wk, Wv, Wo):
    b, lq, dim_q = q.shape
    q2 = q.reshape(lq, dim_q)
    k2 = k.reshape(lq, dim_q)
    v2 = v.reshape(lq, dim_q)

    qt, kt, vt = pl.pallas_call(
        _proj_kernel,
        grid=(1,),
        in_specs=[pl.BlockSpec((LQ, DIM), lambda i: (0, 0))] * 3
        + [pl.BlockSpec((DIM, DIM), lambda i: (0, 0))] * 3,
        out_specs=[pl.BlockSpec((DIM, LQ), lambda i: (0, 0))] * 3,
        out_shape=[jax.ShapeDtypeStruct((DIM, LQ), jnp.float32)] * 3,
    )(q2, k2, v2, Wq, Wk, Wv)

    out = pl.pallas_call(
        _band_kernel,
        grid=(HEADS,),
        in_specs=[
            pl.BlockSpec((DQK, LQ), lambda h: (h, 0)),
            pl.BlockSpec((DQK, LQ), lambda h: (h, 0)),
            pl.BlockSpec((DV, LQ), lambda h: (h, 0)),
            pl.BlockSpec((DIM, HEADS * DV), lambda h: (0, 0)),
        ],
        out_specs=pl.BlockSpec((LQ, DIM), lambda h: (0, 0)),
        out_shape=jax.ShapeDtypeStruct((LQ, DIM), jnp.float32),
        scratch_shapes=[pltpu.VMEM((HEADS * DV, LQ), jnp.float32)],
    )(qt, kt, vt, Wo)

    return out.reshape(b, lq, DIM)
